# Initial kernel scaffold; baseline (speedup 1.0000x reference)
import jax
import jax.numpy as jnp
from jax.experimental import pallas as pl


def kernel(x, edge_index, y_one_hot, train_mask):
    raise NotImplementedError("write your pallas kernel here")


# speedup vs baseline: 13.4899x; 13.4899x over previous
import functools

import jax
import jax.numpy as jnp
from jax import lax
from jax.experimental import pallas as pl
from jax.experimental.pallas import tpu as pltpu
from jax.experimental.pallas import tpu_sc as plsc

N = 10000
E = 320000
D = 128
C = 16
ALPHA = 0.1
REG = 1e-05

NW = 32
NP = 320
N_PAD = NW * NP
CH = 128
E_CHUNKS = E // CH
EH_CHUNKS = (E // 2) // CH
NPT = N_PAD // 16


def _rsqrt16(v):
    i = lax.bitcast_convert_type(v, jnp.int32)
    i = jnp.int32(0x5F3759DF) - (i >> 1)
    y = lax.bitcast_convert_type(i, jnp.float32)
    for _ in range(3):
        y = y * (1.5 - 0.5 * v * y * y)
    return y


def _deg_dinv_s0_body(col_hbm, x_hbm, dinv_hbm, s0_hbm,
                      deg_sh, zbuf, ones_v, idx_v, degv, xv, sem):
    cid = lax.axis_index("c")
    sid = lax.axis_index("s")
    wid = cid * 16 + sid

    for g in range(NPT // 16):
        zbuf[pl.ds(16 * g, 16)] = jnp.zeros((16,), jnp.float32)
    pltpu.sync_copy(zbuf, deg_sh.at[pl.ds(sid * NPT, NPT)])
    for g in range(CH // 16):
        ones_v[pl.ds(16 * g, 16)] = jnp.full((16,), 1.0, jnp.float32)
    plsc.subcore_barrier()

    def deg_step(j, _):
        k = sid + 16 * j

        @pl.when(k < E_CHUNKS)
        def _():
            pltpu.sync_copy(col_hbm.at[pl.ds(k * CH, CH)], idx_v)
            pltpu.sync_copy(ones_v, deg_sh.at[idx_v], add=True)
        return _

    lax.fori_loop(0, (E_CHUNKS + 15) // 16, deg_step, None)
    plsc.subcore_barrier()

    base = wid * NP
    pltpu.sync_copy(deg_sh.at[pl.ds(base, NP)], degv)
    for g in range(NP // 16):
        ids = base + 16 * g + lax.iota(jnp.int32, 16)
        real = ids < N
        dv = degv[pl.ds(16 * g, 16)] + jnp.where(real, 1.0, 0.0)
        degv[pl.ds(16 * g, 16)] = jnp.where(real, _rsqrt16(dv), 0.0)
    pltpu.sync_copy(degv, dinv_hbm.at[pl.ds(base, NP)])

    pltpu.sync_copy(x_hbm.at[pl.ds(base, NP)], xv)

    def scale_group(g, _):
        dvec = degv[pl.ds(16 * g, 16)]
        for l in range(16):
            dv = dvec[l]
            r = 16 * g + l
            for j in range(D // 16):
                xv[r, pl.ds(16 * j, 16)] = xv[r, pl.ds(16 * j, 16)] * dv
        return _

    lax.fori_loop(0, NP // 16, scale_group, None)
    pltpu.sync_copy(xv, s0_hbm.at[pl.ds(base, NP)])


def _deg_dinv_s0(col, x_pad):
    mesh = plsc.VectorSubcoreMesh(core_axis_name="c", subcore_axis_name="s")
    return pl.kernel(
        _deg_dinv_s0_body,
        out_type=(
            jax.ShapeDtypeStruct((N_PAD,), jnp.float32),
            jax.ShapeDtypeStruct((N_PAD, D), jnp.float32),
        ),
        mesh=mesh,
        scratch_types=[
            pltpu.VMEM_SHARED((N_PAD,), jnp.float32),
            pltpu.VMEM((NPT,), jnp.float32),
            pltpu.VMEM((CH,), jnp.float32),
            pltpu.VMEM((CH,), jnp.int32),
            pltpu.VMEM((NP,), jnp.float32),
            pltpu.VMEM((NP, D), jnp.float32),
            pltpu.SemaphoreType.DMA,
        ],
    )(col, x_pad)


def _hop_body(s_hbm, row_hbm, col_hbm, tp_hbm,
              acc_sh, ri_v, ci_v, rows_v, sem):
    cid = lax.axis_index("c")
    sid = lax.axis_index("s")

    def zrow(r, _):
        for g in range(D // 16):
            rows_v[r, pl.ds(16 * g, 16)] = jnp.zeros((16,), jnp.float32)
        return _

    lax.fori_loop(0, CH, zrow, None)
    for b in range(NPT // CH):
        pltpu.sync_copy(rows_v, acc_sh.at[pl.ds(sid * NPT + b * CH, CH)])
    plsc.subcore_barrier()

    ebase = cid * (E // 2)

    def edge_step(j, _):
        k = sid + 16 * j

        @pl.when(k < EH_CHUNKS)
        def _():
            pltpu.sync_copy(row_hbm.at[pl.ds(ebase + k * CH, CH)], ri_v)
            pltpu.sync_copy(col_hbm.at[pl.ds(ebase + k * CH, CH)], ci_v)
            pltpu.async_copy(s_hbm.at[ri_v], rows_v, sem).wait()
            pltpu.sync_copy(rows_v, acc_sh.at[ci_v], add=True)
        return _

    lax.fori_loop(0, (EH_CHUNKS + 15) // 16, edge_step, None)
    plsc.subcore_barrier()

    pltpu.sync_copy(acc_sh.at[pl.ds(sid * NPT, NPT)],
                    tp_hbm.at[cid, pl.ds(sid * NPT, NPT)])


def _hop(s, row, col):
    mesh = plsc.VectorSubcoreMesh(core_axis_name="c", subcore_axis_name="s")
    return pl.kernel(
        _hop_body,
        out_type=jax.ShapeDtypeStruct((2, N_PAD, D), jnp.float32),
        mesh=mesh,
        scratch_types=[
            pltpu.VMEM_SHARED((N_PAD, D), jnp.float32),
            pltpu.VMEM((CH,), jnp.int32),
            pltpu.VMEM((CH,), jnp.int32),
            pltpu.VMEM((CH, D), jnp.float32),
            pltpu.SemaphoreType.DMA,
        ],
    )(s, row, col)


BN = 2048
BN2 = 1000


def _combine1_body(tp0, tp1, s, x, dinv, s_next):
    t = tp0[...] + tp1[...] + s[...]
    dv = dinv[...]
    out = (1.0 - ALPHA) * (dv * t) + ALPHA * x[...]
    s_next[...] = dv * out


def _combine1(tp, s, x_pad, dinv_col):
    grid = (N_PAD // BN,)
    return pl.pallas_call(
        _combine1_body,
        grid=grid,
        in_specs=[
            pl.BlockSpec((BN, D), lambda i: (i, 0)),
            pl.BlockSpec((BN, D), lambda i: (i, 0)),
            pl.BlockSpec((BN, D), lambda i: (i, 0)),
            pl.BlockSpec((BN, D), lambda i: (i, 0)),
            pl.BlockSpec((BN, 1), lambda i: (i, 0)),
        ],
        out_specs=pl.BlockSpec((BN, D), lambda i: (i, 0)),
        out_shape=jax.ShapeDtypeStruct((N_PAD, D), jnp.float32),
    )(tp[0], tp[1], s, x_pad, dinv_col)


def _combine2_body(tp0, tp1, s, x, dinv, mask, yb, xg_out, g_out, r_out,
                   acc_g, acc_r):
    i = pl.program_id(0)

    @pl.when(i == 0)
    def _():
        acc_g[...] = jnp.zeros_like(acc_g)
        acc_r[...] = jnp.zeros_like(acc_r)

    t = tp0[...] + tp1[...] + s[...]
    dv = dinv[...]
    xg = (1.0 - ALPHA) * (dv * t) + ALPHA * x[...]
    xg_out[...] = xg
    xm = xg * mask[...]
    acc_g[...] += lax.dot_general(xm, xg, (((0,), (0,)), ((), ())),
                                  preferred_element_type=jnp.float32)
    acc_r[...] += lax.dot_general(xm, yb[...], (((0,), (0,)), ((), ())),
                                  preferred_element_type=jnp.float32)

    @pl.when(i == N_PAD // BN - 1)
    def _():
        rows = lax.broadcasted_iota(jnp.int32, (D, D), 0)
        cols = lax.broadcasted_iota(jnp.int32, (D, D), 1)
        eye = jnp.where(rows == cols, jnp.float32(REG), jnp.float32(0.0))
        g_out[...] = acc_g[...] + eye
        r_out[...] = acc_r[...]


def _combine2(tp, s, x_pad, dinv_col, mask_col, y_pad):
    grid = (N_PAD // BN,)
    return pl.pallas_call(
        _combine2_body,
        grid=grid,
        in_specs=[
            pl.BlockSpec((BN, D), lambda i: (i, 0)),
            pl.BlockSpec((BN, D), lambda i: (i, 0)),
            pl.BlockSpec((BN, D), lambda i: (i, 0)),
            pl.BlockSpec((BN, D), lambda i: (i, 0)),
            pl.BlockSpec((BN, 1), lambda i: (i, 0)),
            pl.BlockSpec((BN, 1), lambda i: (i, 0)),
            pl.BlockSpec((BN, C), lambda i: (i, 0)),
        ],
        out_specs=[
            pl.BlockSpec((BN, D), lambda i: (i, 0)),
            pl.BlockSpec((D, D), lambda i: (0, 0)),
            pl.BlockSpec((D, C), lambda i: (0, 0)),
        ],
        out_shape=[
            jax.ShapeDtypeStruct((N_PAD, D), jnp.float32),
            jax.ShapeDtypeStruct((D, D), jnp.float32),
            jax.ShapeDtypeStruct((D, C), jnp.float32),
        ],
        scratch_shapes=[
            pltpu.VMEM((D, D), jnp.float32),
            pltpu.VMEM((D, C), jnp.float32),
        ],
    )(tp[0], tp[1], s, x_pad, dinv_col, mask_col, y_pad)


def _solve_predict_body(g_ref, r_ref, xg, yp, sol):
    i = pl.program_id(0)

    @pl.when(i == 0)
    def _():
        a = g_ref[...]
        aabs = jnp.abs(a)
        n1 = jnp.max(jnp.sum(aabs, axis=0))
        ninf = jnp.max(jnp.sum(aabs, axis=1))
        rows = lax.broadcasted_iota(jnp.int32, (D, D), 0)
        cols = lax.broadcasted_iota(jnp.int32, (D, D), 1)
        two_i = jnp.where(rows == cols, jnp.float32(2.0), jnp.float32(0.0))
        x0 = a * (1.0 / (n1 * ninf))

        xinv = x0
        for _ in range(24):
            ax = lax.dot_general(a, xinv, (((1,), (0,)), ((), ())),
                                 preferred_element_type=jnp.float32, precision=lax.Precision.HIGHEST)
            xinv = lax.dot_general(xinv, two_i - ax, (((1,), (0,)), ((), ())),
                                   preferred_element_type=jnp.float32, precision=lax.Precision.HIGHEST)
        sol[...] = lax.dot_general(xinv, r_ref[...], (((1,), (0,)), ((), ())),
                                   preferred_element_type=jnp.float32)

    yp[...] = lax.dot_general(xg[...], sol[...], (((1,), (0,)), ((), ())),
                              preferred_element_type=jnp.float32)


def _solve_predict(g_mat, r_mat, xg):
    grid = (N // BN2,)
    return pl.pallas_call(
        _solve_predict_body,
        grid=grid,
        in_specs=[
            pl.BlockSpec((D, D), lambda i: (0, 0)),
            pl.BlockSpec((D, C), lambda i: (0, 0)),
            pl.BlockSpec((BN2, D), lambda i: (i, 0)),
        ],
        out_specs=pl.BlockSpec((BN2, C), lambda i: (i, 0)),
        out_shape=jax.ShapeDtypeStruct((N, C), jnp.float32),
        scratch_shapes=[pltpu.VMEM((D, C), jnp.float32)],
    )(g_mat, r_mat, xg)


def kernel(x, edge_index, y_one_hot, train_mask):
    row = edge_index[0]
    col = edge_index[1]
    x_pad = jnp.pad(x, ((0, N_PAD - N), (0, 0)))
    y_pad = jnp.pad(y_one_hot, ((0, N_PAD - N), (0, 0)))
    mask_col = jnp.pad(train_mask.astype(jnp.float32), (0, N_PAD - N))[:, None]

    dinv, s0 = _deg_dinv_s0(col, x_pad)
    dinv_col = dinv[:, None]

    tp1 = _hop(s0, row, col)
    s1 = _combine1(tp1, s0, x_pad, dinv_col)
    tp2 = _hop(s1, row, col)
    xg, g_mat, r_mat = _combine2(tp2, s1, x_pad, dinv_col, mask_col, y_pad)
    return _solve_predict(g_mat, r_mat, xg)
